# baseline (speedup 1.0000x reference)
import functools

import jax
import jax.numpy as jnp
from jax import lax
from jax.experimental import pallas as pl
from jax.experimental.pallas import tpu as pltpu
from jax.experimental.pallas import tpu_sc as plsc

_MIN_SPEED = 0.5
_MAX_SPEED = 2.0

_NUM_CORES = 2
_NUM_SUBCORES = 16
_NW = _NUM_CORES * _NUM_SUBCORES
_L = 16


def _make_sc_call(N, C, T, ROW):
    UL = T * ROW
    NG = (T + _L - 1) // _L
    TP = NG * _L
    UNITS = N * C
    UPW = UNITS // _NW
    assert UNITS % _NW == 0 and UL % 8 == 0 and (2 * TP) % 8 == 0

    mesh = plsc.VectorSubcoreMesh(
        core_axis_name="c", subcore_axis_name="s",
        num_cores=_NUM_CORES, num_subcores=_NUM_SUBCORES)

    @functools.partial(
        pl.kernel,
        out_type=jax.ShapeDtypeStruct((UNITS * UL,), jnp.float32),
        mesh=mesh,
        scratch_types=[
            pltpu.VMEM((UL,), jnp.float32),
            pltpu.VMEM((TP * ROW,), jnp.float32),
            pltpu.VMEM((2 * TP,), jnp.float32),
        ],
        compiler_params=pltpu.CompilerParams(needs_layout_passes=False),
    )
    def sc_call(x_hbm, lw_hbm, out_hbm, in_v, out_v, lw_v):
        wid = lax.axis_index("s") * _NUM_CORES + lax.axis_index("c")
        iota = lax.iota(jnp.int32, _L)

        def unit_body(k, carry):
            u = wid * UPW + k
            n = u // C
            pltpu.sync_copy(lw_hbm.at[pl.ds(n * (2 * TP), 2 * TP)], lw_v)
            pltpu.sync_copy(x_hbm.at[pl.ds(u * UL, UL)], in_v)

            def group_body(g, gcarry):
                lf = lw_v[pl.ds(g * _L, _L)]
                w = lw_v[pl.ds(TP + g * _L, _L)]
                li = lf.astype(jnp.int32)
                ri = jnp.minimum(li + 1, T - 1)
                bl = li * ROW
                br = ri * ROW
                ob = (g * _L + iota) * ROW
                for j in range(ROW):
                    a = plsc.load_gather(in_v, [bl])
                    b = plsc.load_gather(in_v, [br])
                    res = a + w * (b - a)
                    plsc.store_scatter(out_v, [ob], res)
                    if j + 1 < ROW:
                        bl = bl + 1
                        br = br + 1
                        ob = ob + 1
                return gcarry

            lax.fori_loop(0, NG, group_body, 0)
            pltpu.sync_copy(out_v.at[pl.ds(0, UL)],
                            out_hbm.at[pl.ds(u * UL, UL)])
            return carry

        lax.fori_loop(0, UPW, unit_body, 0)

    return sc_call, TP


def kernel(x):
    N, C, T, V, M = x.shape
    ROW = V * M

    skey = jax.random.key(42)
    speed = (jax.random.uniform(skey, (N,), dtype=jnp.float32)
             * (_MAX_SPEED - _MIN_SPEED) + _MIN_SPEED)

    sc_call, TP = _make_sc_call(N, C, T, ROW)

    t = jnp.arange(T, dtype=jnp.float32)[None, :]
    t_new = jnp.clip(t / speed[:, None], 0.0, float(T - 1))
    lf = jnp.floor(t_new)
    w = t_new - lf
    lw = jnp.zeros((N, 2 * TP), jnp.float32)
    lw = lw.at[:, :T].set(lf).at[:, TP:TP + T].set(w)

    out_flat = sc_call(x.reshape(-1), lw.reshape(-1))
    return out_flat.reshape(N, C, T, V, M), speed
